# baseline (speedup 1.0000x reference)
import functools

import jax
import jax.numpy as jnp
from jax import lax
from jax.experimental import pallas as pl
from jax.experimental.pallas import tpu as pltpu
from jax.experimental.pallas import tpu_sc as plsc

N = 2048
D = 768
E = 64
F = 3072
C = 40
CP = C
ZERO_ROW = E * CP

TB = 256
NBLK = N // TB

FB = 768
NF = F // FB

NC, NS = 2, 16
NW = NC * NS
TPW = N // NW


def _router_body(x_ref, gw_ref, sx_ref, sy_ref, carry_ref):
    i = pl.program_id(0)

    @pl.when(i == 0)
    def _():
        carry_ref[...] = jnp.zeros_like(carry_ref)

    logits = jnp.dot(x_ref[...], gw_ref[...],
                     preferred_element_type=jnp.float32)
    m = jnp.max(logits, axis=1, keepdims=True)
    cols = lax.broadcasted_iota(jnp.int32, (TB, E), 1)
    eid = jnp.min(jnp.where(logits == m, cols, jnp.int32(2 ** 30)),
                  axis=1, keepdims=True)
    maskf = (cols == eid).astype(jnp.float32)

    r = lax.broadcasted_iota(jnp.int32, (TB, TB), 0)
    c = lax.broadcasted_iota(jnp.int32, (TB, TB), 1)
    lt = (c < r).astype(jnp.float32)
    excl = jnp.dot(lt, maskf, preferred_element_type=jnp.float32)

    carry = carry_ref[0:1, :]
    rank = jnp.sum((excl + carry) * maskf, axis=1,
                   keepdims=True).astype(jnp.int32)
    carry_ref[0:1, :] = carry + jnp.sum(maskf, axis=0, keepdims=True)

    kept = rank < C
    trash = ZERO_ROW + jnp.minimum(rank - C, CP - 1)
    slot_x = jnp.where(kept, eid * CP + rank, trash)
    slot_y = jnp.where(kept, eid * CP + rank, jnp.int32(ZERO_ROW))
    sx_ref[...] = slot_x
    sy_ref[...] = slot_y


def _router(x2, gate_w):
    sx, sy = pl.pallas_call(
        _router_body,
        grid=(NBLK,),
        in_specs=[
            pl.BlockSpec((TB, D), lambda i: (i, 0)),
            pl.BlockSpec((D, E), lambda i: (0, 0)),
        ],
        out_specs=[
            pl.BlockSpec((TB, 1), lambda i: (i, 0)),
            pl.BlockSpec((TB, 1), lambda i: (i, 0)),
        ],
        out_shape=[
            jax.ShapeDtypeStruct((N, 1), jnp.int32),
            jax.ShapeDtypeStruct((N, 1), jnp.int32),
        ],
        scratch_shapes=[pltpu.VMEM((8, E), jnp.float32)],
    )(x2, gate_w)
    return sx.reshape(N), sy.reshape(N)


@functools.lru_cache(maxsize=None)
def _make_dispatch():
    mesh = plsc.VectorSubcoreMesh(core_axis_name="c", subcore_axis_name="s")

    @functools.partial(
        pl.kernel,
        mesh=mesh,
        out_type=jax.ShapeDtypeStruct(((E + 1) * CP, D), jnp.float32),
        scratch_types=[
            pltpu.VMEM((TPW,), jnp.int32),
            pltpu.VMEM((TPW, D), jnp.float32),
            pltpu.SemaphoreType.DMA,
        ],
    )
    def dispatch(x_hbm, slot_hbm, xb_hbm, idx_v, rows_v, sem):
        wid = lax.axis_index("s") * NC + lax.axis_index("c")
        base = wid * TPW
        pltpu.sync_copy(slot_hbm.at[pl.ds(base, TPW)], idx_v)
        pltpu.sync_copy(x_hbm.at[pl.ds(base, TPW)], rows_v)
        pltpu.async_copy(rows_v, xb_hbm.at[idx_v], sem).wait()

    return dispatch


@functools.lru_cache(maxsize=None)
def _make_combine():
    mesh = plsc.VectorSubcoreMesh(core_axis_name="c", subcore_axis_name="s")

    @functools.partial(
        pl.kernel,
        mesh=mesh,
        out_type=jax.ShapeDtypeStruct((N, D), jnp.float32),
        scratch_types=[
            pltpu.VMEM((TPW,), jnp.int32),
            pltpu.VMEM((TPW, D), jnp.float32),
            pltpu.SemaphoreType.DMA,
        ],
    )
    def combine(yb_hbm, slot_hbm, out_hbm, idx_v, rows_v, sem):
        wid = lax.axis_index("s") * NC + lax.axis_index("c")
        base = wid * TPW
        pltpu.sync_copy(slot_hbm.at[pl.ds(base, TPW)], idx_v)
        pltpu.async_copy(yb_hbm.at[idx_v], rows_v, sem).wait()
        pltpu.sync_copy(rows_v, out_hbm.at[pl.ds(base, TPW)])

    return combine


def _gelu_exact(x):
    return 0.5 * x * (1.0 + lax.erf(x * 0.7071067811865476))


def _expert_body(xb_ref, cfc_ref, fcb_ref, cpj_ref, pjb_ref, out_ref):
    e = pl.program_id(0)

    h = jnp.dot(xb_ref[0], cfc_ref[0],
                preferred_element_type=jnp.float32) + fcb_ref[0]
    h = _gelu_exact(h)
    y = jnp.dot(h, cpj_ref[0], preferred_element_type=jnp.float32) + pjb_ref[0]
    out_ref[0] = jnp.where(e == E, jnp.float32(0.0), y)


def _experts(xb, c_fc, c_proj, fc_bias, proj_bias):
    yb = pl.pallas_call(
        _expert_body,
        grid=(E + 1,),
        in_specs=[
            pl.BlockSpec((1, CP, D), lambda e: (e, 0, 0)),
            pl.BlockSpec((1, D, F), lambda e: (jnp.minimum(e, E - 1), 0, 0)),
            pl.BlockSpec((1, 1, F), lambda e: (jnp.minimum(e, E - 1), 0, 0)),
            pl.BlockSpec((1, F, D), lambda e: (jnp.minimum(e, E - 1), 0, 0)),
            pl.BlockSpec((1, 1, D), lambda e: (jnp.minimum(e, E - 1), 0, 0)),
        ],
        out_specs=pl.BlockSpec((1, CP, D), lambda e: (e, 0, 0)),
        out_shape=jax.ShapeDtypeStruct((E + 1, CP, D), jnp.float32),
    )(xb.reshape(E + 1, CP, D), c_fc, fc_bias, c_proj, proj_bias)
    return yb.reshape((E + 1) * CP, D)


def kernel(x, gate_w, c_fc, c_proj, fc_bias, proj_bias):
    x2 = x.reshape(N, D)
    slot_x, slot_y = _router(x2, gate_w)
    xb = _make_dispatch()(x2, slot_x)
    yb = _experts(xb, c_fc, c_proj, fc_bias, proj_bias)
    out = _make_combine()(yb, slot_y)
    return out.reshape(x.shape)


# speedup vs baseline: 1.0039x; 1.0039x over previous
import functools

import jax
import jax.numpy as jnp
from jax import lax
from jax.experimental import pallas as pl
from jax.experimental.pallas import tpu as pltpu
from jax.experimental.pallas import tpu_sc as plsc

N = 2048
D = 768
E = 64
F = 3072
C = 40
CP = C
ZERO_ROW = E * CP

TB = 256
NBLK = N // TB

FB = 768
NF = F // FB

NC, NS = 2, 16
NW = NC * NS
TPW = N // NW


def _router_body(x_ref, gw_ref, sx_ref, sy_ref, carry_ref):
    i = pl.program_id(0)

    @pl.when(i == 0)
    def _():
        carry_ref[...] = jnp.zeros_like(carry_ref)

    logits = jnp.dot(x_ref[...], gw_ref[...],
                     preferred_element_type=jnp.float32)
    m = jnp.max(logits, axis=1, keepdims=True)
    cols = lax.broadcasted_iota(jnp.int32, (TB, E), 1)
    eid = jnp.min(jnp.where(logits == m, cols, jnp.int32(2 ** 30)),
                  axis=1, keepdims=True)
    maskf = (cols == eid).astype(jnp.float32)

    r = lax.broadcasted_iota(jnp.int32, (TB, TB), 0)
    c = lax.broadcasted_iota(jnp.int32, (TB, TB), 1)
    lt = (c < r).astype(jnp.float32)
    excl = jnp.dot(lt, maskf, preferred_element_type=jnp.float32)

    carry = carry_ref[0:1, :]
    rank = jnp.sum((excl + carry) * maskf, axis=1,
                   keepdims=True).astype(jnp.int32)
    carry_ref[0:1, :] = carry + jnp.sum(maskf, axis=0, keepdims=True)

    kept = rank < C
    trash = ZERO_ROW + jnp.minimum(rank - C, CP - 1)
    slot_x = jnp.where(kept, eid * CP + rank, trash)
    slot_y = jnp.where(kept, eid * CP + rank, jnp.int32(ZERO_ROW))
    sx_ref[...] = slot_x
    sy_ref[...] = slot_y


def _router(x2, gate_w):
    sx, sy = pl.pallas_call(
        _router_body,
        grid=(NBLK,),
        in_specs=[
            pl.BlockSpec((TB, D), lambda i: (i, 0)),
            pl.BlockSpec((D, E), lambda i: (0, 0)),
        ],
        out_specs=[
            pl.BlockSpec((TB, 1), lambda i: (i, 0)),
            pl.BlockSpec((TB, 1), lambda i: (i, 0)),
        ],
        out_shape=[
            jax.ShapeDtypeStruct((N, 1), jnp.int32),
            jax.ShapeDtypeStruct((N, 1), jnp.int32),
        ],
        scratch_shapes=[pltpu.VMEM((8, E), jnp.float32)],
    )(x2, gate_w)
    return sx.reshape(N), sy.reshape(N)


@functools.lru_cache(maxsize=None)
def _make_dispatch():
    mesh = plsc.VectorSubcoreMesh(core_axis_name="c", subcore_axis_name="s")

    @functools.partial(
        pl.kernel,
        mesh=mesh,
        out_type=jax.ShapeDtypeStruct(((E + 1) * CP, D), jnp.float32),
        scratch_types=[
            pltpu.VMEM((TPW,), jnp.int32),
            pltpu.VMEM((TPW, D), jnp.float32),
            pltpu.SemaphoreType.DMA,
            pltpu.SemaphoreType.DMA,
        ],
    )
    def dispatch(x_hbm, slot_hbm, xb_hbm, idx_v, rows_v, s1, s2):
        wid = lax.axis_index("s") * NC + lax.axis_index("c")
        base = wid * TPW
        la = pltpu.async_copy(x_hbm.at[pl.ds(base, TPW)], rows_v, s1)
        ia = pltpu.async_copy(slot_hbm.at[pl.ds(base, TPW)], idx_v, s2)
        la.wait()
        ia.wait()
        pltpu.async_copy(rows_v, xb_hbm.at[idx_v], s1).wait()

    return dispatch


@functools.lru_cache(maxsize=None)
def _make_combine():
    mesh = plsc.VectorSubcoreMesh(core_axis_name="c", subcore_axis_name="s")

    @functools.partial(
        pl.kernel,
        mesh=mesh,
        out_type=jax.ShapeDtypeStruct((N, D), jnp.float32),
        scratch_types=[
            pltpu.VMEM((TPW,), jnp.int32),
            pltpu.VMEM((TPW, D), jnp.float32),
            pltpu.SemaphoreType.DMA,
        ],
    )
    def combine(yb_hbm, slot_hbm, out_hbm, idx_v, rows_v, sem):
        wid = lax.axis_index("s") * NC + lax.axis_index("c")
        base = wid * TPW
        pltpu.sync_copy(slot_hbm.at[pl.ds(base, TPW)], idx_v)
        pltpu.async_copy(yb_hbm.at[idx_v], rows_v, sem).wait()
        pltpu.sync_copy(rows_v, out_hbm.at[pl.ds(base, TPW)])

    return combine


def _gelu_exact(x):
    return 0.5 * x * (1.0 + lax.erf(x * 0.7071067811865476))


def _expert_body(xb_ref, cfc_ref, fcb_ref, cpj_ref, pjb_ref, out_ref):
    e = pl.program_id(0)

    h = jnp.dot(xb_ref[0], cfc_ref[0],
                preferred_element_type=jnp.float32) + fcb_ref[0]
    h = _gelu_exact(h)
    y = jnp.dot(h, cpj_ref[0], preferred_element_type=jnp.float32) + pjb_ref[0]
    out_ref[0] = jnp.where(e == E, jnp.float32(0.0), y)


def _experts(xb, c_fc, c_proj, fc_bias, proj_bias):
    yb = pl.pallas_call(
        _expert_body,
        grid=(E + 1,),
        in_specs=[
            pl.BlockSpec((1, CP, D), lambda e: (e, 0, 0)),
            pl.BlockSpec((1, D, F), lambda e: (jnp.minimum(e, E - 1), 0, 0)),
            pl.BlockSpec((1, 1, F), lambda e: (jnp.minimum(e, E - 1), 0, 0)),
            pl.BlockSpec((1, F, D), lambda e: (jnp.minimum(e, E - 1), 0, 0)),
            pl.BlockSpec((1, 1, D), lambda e: (jnp.minimum(e, E - 1), 0, 0)),
        ],
        out_specs=pl.BlockSpec((1, CP, D), lambda e: (e, 0, 0)),
        out_shape=jax.ShapeDtypeStruct((E + 1, CP, D), jnp.float32),
    )(xb.reshape(E + 1, CP, D), c_fc, fc_bias, c_proj, proj_bias)
    return yb.reshape((E + 1) * CP, D)


def kernel(x, gate_w, c_fc, c_proj, fc_bias, proj_bias):
    x2 = x.reshape(N, D)
    slot_x, slot_y = _router(x2, gate_w)
    xb = _make_dispatch()(x2, slot_x)
    yb = _experts(xb, c_fc, c_proj, fc_bias, proj_bias)
    out = _make_combine()(yb, slot_y)
    return out.reshape(x.shape)
